# baseline (speedup 1.0000x reference)
import functools

import jax
import jax.numpy as jnp
from jax import lax
from jax.experimental import pallas as pl
from jax.experimental.pallas import tpu as pltpu
from jax.experimental.pallas import tpu_sc as plsc

VOCAB = 1000000
EMBED = 64
BATCH = 16384
C = 6

NC, NS = 2, 16
NW = NC * NS
BPW = BATCH // NW
CHUNK = 64
NCHUNK = BPW // CHUNK
CROWS = CHUNK * C
NV = EMBED // 16


@functools.cache
def _make_sc_kernel():
    mesh = plsc.VectorSubcoreMesh(core_axis_name="c", subcore_axis_name="s")

    @functools.partial(
        pl.kernel,
        mesh=mesh,
        out_type=jax.ShapeDtypeStruct((BATCH * C,), jnp.float32),
        compiler_params=pltpu.CompilerParams(needs_layout_passes=False,
                                             use_tc_tiling_on_sc=False),
        scratch_types=[
            pltpu.VMEM((BPW,), jnp.int32),
            pltpu.VMEM((BPW * C,), jnp.int32),
            pltpu.VMEM((CHUNK, EMBED), jnp.float32),
            pltpu.VMEM((CHUNK, EMBED), jnp.float32),
            pltpu.VMEM((CROWS, EMBED), jnp.float32),
            pltpu.VMEM((CROWS, EMBED), jnp.float32),
            pltpu.VMEM((BPW * C,), jnp.float32),
            pltpu.SemaphoreType.DMA,
            pltpu.SemaphoreType.DMA,
        ],
    )
    def sc_kernel(tgt_idx_hbm, ctx_idx_hbm, tgt_table_hbm, ctx_table_hbm,
                  out_hbm, tgt_idx_v, ctx_idx_v, tgt_rows0, tgt_rows1,
                  ctx_rows0, ctx_rows1, out_v, sem0, sem1):
        wid = lax.axis_index("s") * NC + lax.axis_index("c")
        base = wid * BPW

        tgt_rows = (tgt_rows0, tgt_rows1)
        ctx_rows = (ctx_rows0, ctx_rows1)
        sems = (sem0, sem1)

        pltpu.sync_copy(tgt_idx_hbm.at[pl.ds(base, BPW)], tgt_idx_v)
        pltpu.sync_copy(ctx_idx_hbm.at[pl.ds(base * C, BPW * C)], ctx_idx_v)

        lane = lax.iota(jnp.int32, 16)
        last_lane = lane == 15

        def fire(j, par):
            hs = [pltpu.async_copy(
                tgt_table_hbm.at[tgt_idx_v.at[pl.ds(j * CHUNK, CHUNK)]],
                tgt_rows[par], sems[par])]
            for k in range(CROWS // 128):
                hs.append(pltpu.async_copy(
                    ctx_table_hbm.at[ctx_idx_v.at[pl.ds(j * CROWS + k * 128,
                                                        128)]],
                    ctx_rows[par].at[pl.ds(k * 128, 128)], sems[par]))
            return hs

        def compute(j, par):
            trows, crows = tgt_rows[par], ctx_rows[par]

            def body(bl, _):
                tv = [trows[bl, pl.ds(16 * v, 16)] for v in range(NV)]
                out_base = (j * CHUNK + bl) * C
                for c in range(C):
                    row = bl * C + c
                    acc = tv[0] * crows[row, pl.ds(0, 16)]
                    for v in range(1, NV):
                        acc += tv[v] * crows[row, pl.ds(16 * v, 16)]
                    total = plsc.cumsum(acc)
                    idx = jnp.full((16,), out_base + c, jnp.int32)
                    plsc.store_scatter(out_v, [idx], total, mask=last_lane)
                return ()

            lax.fori_loop(0, CHUNK, body, ())

        pending = [None, None]
        pending[0] = fire(0, 0)
        for j in range(NCHUNK):
            par = j % 2
            if j + 1 < NCHUNK:
                pending[1 - par] = fire(j + 1, 1 - par)
            for h in pending[par]:
                h.wait()
            compute(j, par)

        pltpu.sync_copy(out_v, out_hbm.at[pl.ds(base * C, BPW * C)])

    return sc_kernel


def kernel(target, context, target_table, context_table):
    tgt_idx = target.reshape(BATCH).astype(jnp.int32)
    ctx_idx = context.reshape(BATCH * C).astype(jnp.int32)
    out = _make_sc_kernel()(tgt_idx, ctx_idx, target_table, context_table)
    return out.reshape(BATCH, C)


# speedup vs baseline: 1.4695x; 1.4695x over previous
import functools

import jax
import jax.numpy as jnp
from jax import lax
from jax.experimental import pallas as pl
from jax.experimental.pallas import tpu as pltpu
from jax.experimental.pallas import tpu_sc as plsc

VOCAB = 1000000
EMBED = 64
BATCH = 16384
C = 6

NC, NS = 2, 16
NW = NC * NS
BPW = BATCH // NW
CHUNK = 64
NCHUNK = BPW // CHUNK
CROWS = CHUNK * C
NV = EMBED // 16
TGROUPS = CHUNK // 16
CGROUPS = CROWS // 16


@functools.cache
def _make_sc_kernel():
    mesh = plsc.VectorSubcoreMesh(core_axis_name="c", subcore_axis_name="s")

    @functools.partial(
        pl.kernel,
        mesh=mesh,
        out_type=jax.ShapeDtypeStruct((BATCH * C,), jnp.float32),
        compiler_params=pltpu.CompilerParams(needs_layout_passes=False),
        scratch_types=[
            pltpu.VMEM((BPW,), jnp.int32),
            pltpu.VMEM((BPW * C,), jnp.int32),
            pltpu.VMEM((CHUNK, EMBED), jnp.float32),
            pltpu.VMEM((CHUNK, EMBED), jnp.float32),
            pltpu.VMEM((CROWS, EMBED), jnp.float32),
            pltpu.VMEM((CROWS, EMBED), jnp.float32),
            pltpu.VMEM((BPW * C,), jnp.float32),
            pltpu.SemaphoreType.DMA,
            pltpu.SemaphoreType.DMA,
        ],
    )
    def sc_kernel(tgt_idx_hbm, ctx_idx_hbm, tgt_table_hbm, ctx_table_hbm,
                  out_hbm, tgt_idx_v, ctx_idx_v, tgt_rows0, tgt_rows1,
                  ctx_rows0, ctx_rows1, out_v, sem0, sem1):
        wid = lax.axis_index("s") * NC + lax.axis_index("c")
        base = wid * BPW

        tgt_rows = (tgt_rows0, tgt_rows1)
        ctx_rows = (ctx_rows0, ctx_rows1)
        sems = (sem0, sem1)

        pltpu.sync_copy(tgt_idx_hbm.at[pl.ds(base, BPW)], tgt_idx_v)
        pltpu.sync_copy(ctx_idx_hbm.at[pl.ds(base * C, BPW * C)], ctx_idx_v)

        lane = lax.iota(jnp.int32, 16)
        last_lane = lane == 15

        def fire(j, par):
            def tgt_body(g, _):
                iv = tgt_idx_v[pl.ds(j * CHUNK + g * 16, 16)]
                for l in range(16):
                    pltpu.async_copy(
                        tgt_table_hbm.at[pl.ds(iv[l], 1)],
                        tgt_rows[par].at[pl.ds(g * 16 + l, 1)], sems[par])
                return ()

            def ctx_body(g, _):
                iv = ctx_idx_v[pl.ds(j * CROWS + g * 16, 16)]
                for l in range(16):
                    pltpu.async_copy(
                        ctx_table_hbm.at[pl.ds(iv[l], 1)],
                        ctx_rows[par].at[pl.ds(g * 16 + l, 1)], sems[par])
                return ()

            lax.fori_loop(0, TGROUPS, tgt_body, ())
            lax.fori_loop(0, CGROUPS, ctx_body, ())

        def drain(par):
            def body(g, _):
                for _l in range(16):
                    pltpu.make_async_copy(
                        tgt_table_hbm.at[pl.ds(0, 1)],
                        tgt_rows[par].at[pl.ds(0, 1)], sems[par]).wait()
                return ()

            lax.fori_loop(0, TGROUPS + CGROUPS, body, ())

        def compute(j, par):
            trows, crows = tgt_rows[par], ctx_rows[par]

            def body(bl, _):
                tv = [trows[bl, pl.ds(16 * v, 16)] for v in range(NV)]
                out_base = (j * CHUNK + bl) * C
                for c in range(C):
                    row = bl * C + c
                    acc = tv[0] * crows[row, pl.ds(0, 16)]
                    for v in range(1, NV):
                        acc += tv[v] * crows[row, pl.ds(16 * v, 16)]
                    total = plsc.cumsum(acc)
                    idx = jnp.full((16,), out_base + c, jnp.int32)
                    plsc.store_scatter(out_v, [idx], total, mask=last_lane)
                return ()

            lax.fori_loop(0, CHUNK, body, ())

        fire(0, 0)
        for j in range(NCHUNK):
            par = j % 2
            if j + 1 < NCHUNK:
                fire(j + 1, 1 - par)
            drain(par)
            compute(j, par)

        pltpu.sync_copy(out_v, out_hbm.at[pl.ds(base * C, BPW * C)])

    return sc_kernel


def kernel(target, context, target_table, context_table):
    tgt_idx = target.reshape(BATCH).astype(jnp.int32)
    ctx_idx = context.reshape(BATCH * C).astype(jnp.int32)
    out = _make_sc_kernel()(tgt_idx, ctx_idx, target_table, context_table)
    return out.reshape(BATCH, C)
